# baseline (speedup 1.0000x reference)
import functools

import jax
import jax.numpy as jnp
from jax.experimental import pallas as pl
from jax.experimental.pallas import tpu as pltpu

NUM_LINKS = 4096
NUM_PATHS = 8192
NUM_OD = 2048
T_MIN = 1.0
X_MAX = 1000.0

_PATH_BLK = 1024
_LINK_BLK = 256


def _pass1_body(x_ref, a_ref, b_ref, d_ref, t_ref, u_ref):
    base = 1.0 + a_ref[...] * (x_ref[...] * (1.0 / X_MAX))
    t = T_MIN * base ** b_ref[...]
    t_ref[...] = t
    u_ref[...] = jnp.dot(t, d_ref[...], preferred_element_type=jnp.float32)


def _pass1(x_hat, alpha, beta, D):
    grid = NUM_PATHS // _PATH_BLK
    t2, u2 = pl.pallas_call(
        _pass1_body,
        grid=(grid,),
        in_specs=[
            pl.BlockSpec((1, NUM_LINKS), lambda j: (0, 0)),
            pl.BlockSpec((1, NUM_LINKS), lambda j: (0, 0)),
            pl.BlockSpec((1, NUM_LINKS), lambda j: (0, 0)),
            pl.BlockSpec((NUM_LINKS, _PATH_BLK), lambda j: (0, j)),
        ],
        out_specs=[
            pl.BlockSpec((1, NUM_LINKS), lambda j: (0, 0)),
            pl.BlockSpec((1, _PATH_BLK), lambda j: (0, j)),
        ],
        out_shape=[
            jax.ShapeDtypeStruct((1, NUM_LINKS), jnp.float32),
            jax.ShapeDtypeStruct((1, NUM_PATHS), jnp.float32),
        ],
    )(x_hat.reshape(1, NUM_LINKS), alpha.reshape(1, NUM_LINKS),
      beta.reshape(1, NUM_LINKS), D)
    return t2.reshape(NUM_LINKS), u2.reshape(NUM_PATHS)


def _pass2_body(d_ref, f_ref, x_ref):
    x_ref[...] = jnp.dot(d_ref[...], f_ref[...],
                         preferred_element_type=jnp.float32)


def _pass2(D, f):
    grid = NUM_LINKS // _LINK_BLK
    x2 = pl.pallas_call(
        _pass2_body,
        grid=(grid,),
        in_specs=[
            pl.BlockSpec((_LINK_BLK, NUM_PATHS), lambda i: (i, 0)),
            pl.BlockSpec((NUM_PATHS, 1), lambda i: (0, 0)),
        ],
        out_specs=pl.BlockSpec((_LINK_BLK, 1), lambda i: (i, 0)),
        out_shape=jax.ShapeDtypeStruct((NUM_LINKS, 1), jnp.float32),
    )(D, f.reshape(NUM_PATHS, 1))
    return x2.reshape(NUM_LINKS)


def _softmax_scaffold(u, q_hat, segment_ids):
    seg_max = jax.ops.segment_max(u, segment_ids, num_segments=NUM_OD)
    e = jnp.exp(u - seg_max[segment_ids])
    seg_sum = jax.ops.segment_sum(e, segment_ids, num_segments=NUM_OD)
    p = e / seg_sum[segment_ids]
    f = q_hat[segment_ids] * p
    return p, f


def kernel(x_hat, alpha, beta, q_hat, D, M, segment_ids):
    t, u = _pass1(x_hat, alpha, beta, D)
    p, f = _softmax_scaffold(u, q_hat, segment_ids)
    x = _pass2(D, f)
    return (x, t, f, p)


# speedup vs baseline: 1.0344x; 1.0344x over previous
import functools

import jax
import jax.numpy as jnp
from jax.experimental import pallas as pl
from jax.experimental.pallas import tpu as pltpu

NUM_LINKS = 4096
NUM_PATHS = 8192
NUM_OD = 2048
T_MIN = 1.0
X_MAX = 1000.0

_PATH_BLK = 1024
_LINK_BLK = 256


_NW = NUM_LINKS // 32


def _pass1_body(x_ref, a_ref, b_ref, d_ref, t_ref, u_ref, dp_ref):
    base = 1.0 + a_ref[...] * (x_ref[...] * (1.0 / X_MAX))
    t = T_MIN * base ** b_ref[...]
    t_ref[...] = t
    d = d_ref[...]
    u_ref[...] = jnp.dot(t, d, preferred_element_type=jnp.float32)
    acc = jnp.zeros((_NW, d.shape[1]), jnp.int32)
    for k in range(32):
        bits = (d[k * _NW:(k + 1) * _NW, :] != 0.0).astype(jnp.int32)
        acc = acc | (bits << k)
    dp_ref[...] = acc


def _pass1(x_hat, alpha, beta, D):
    grid = NUM_PATHS // _PATH_BLK
    t2, u2, dpack = pl.pallas_call(
        _pass1_body,
        grid=(grid,),
        in_specs=[
            pl.BlockSpec((1, NUM_LINKS), lambda j: (0, 0)),
            pl.BlockSpec((1, NUM_LINKS), lambda j: (0, 0)),
            pl.BlockSpec((1, NUM_LINKS), lambda j: (0, 0)),
            pl.BlockSpec((NUM_LINKS, _PATH_BLK), lambda j: (0, j)),
        ],
        out_specs=[
            pl.BlockSpec((1, NUM_LINKS), lambda j: (0, 0)),
            pl.BlockSpec((1, _PATH_BLK), lambda j: (0, j)),
            pl.BlockSpec((_NW, _PATH_BLK), lambda j: (0, j)),
        ],
        out_shape=[
            jax.ShapeDtypeStruct((1, NUM_LINKS), jnp.float32),
            jax.ShapeDtypeStruct((1, NUM_PATHS), jnp.float32),
            jax.ShapeDtypeStruct((_NW, NUM_PATHS), jnp.int32),
        ],
    )(x_hat.reshape(1, NUM_LINKS), alpha.reshape(1, NUM_LINKS),
      beta.reshape(1, NUM_LINKS), D)
    return t2.reshape(NUM_LINKS), u2.reshape(NUM_PATHS), dpack


def _pass2_body(dp_ref, f_ref, x_ref):
    j = pl.program_id(0)

    @pl.when(j == 0)
    def _init():
        x_ref[...] = jnp.zeros_like(x_ref)

    dp = dp_ref[...]
    fch = f_ref[...]
    for k in range(32):
        mask = ((dp >> k) & 1).astype(jnp.float32)
        x_ref[k * _NW:(k + 1) * _NW, :] += jnp.dot(
            mask, fch, preferred_element_type=jnp.float32)


def _pass2(dpack, f):
    grid = NUM_PATHS // _PATH_BLK
    x2 = pl.pallas_call(
        _pass2_body,
        grid=(grid,),
        in_specs=[
            pl.BlockSpec((_NW, _PATH_BLK), lambda j: (0, j)),
            pl.BlockSpec((_PATH_BLK, 1), lambda j: (j, 0)),
        ],
        out_specs=pl.BlockSpec((NUM_LINKS, 1), lambda j: (0, 0)),
        out_shape=jax.ShapeDtypeStruct((NUM_LINKS, 1), jnp.float32),
    )(dpack, f.reshape(NUM_PATHS, 1))
    return x2.reshape(NUM_LINKS)


def _softmax_scaffold(u, q_hat, segment_ids):
    seg_max = jax.ops.segment_max(u, segment_ids, num_segments=NUM_OD)
    e = jnp.exp(u - seg_max[segment_ids])
    seg_sum = jax.ops.segment_sum(e, segment_ids, num_segments=NUM_OD)
    p = e / seg_sum[segment_ids]
    f = q_hat[segment_ids] * p
    return p, f


def kernel(x_hat, alpha, beta, q_hat, D, M, segment_ids):
    t, u, dpack = _pass1(x_hat, alpha, beta, D)
    p, f = _softmax_scaffold(u, q_hat, segment_ids)
    x = _pass2(dpack, f)
    return (x, t, f, p)


# speedup vs baseline: 3.4241x; 3.3104x over previous
import functools

import jax
import jax.numpy as jnp
from jax import lax
from jax.experimental import pallas as pl
from jax.experimental.pallas import tpu as pltpu
from jax.experimental.pallas import tpu_sc as plsc

NUM_LINKS = 4096
NUM_PATHS = 8192
NUM_OD = 2048
T_MIN = 1.0
X_MAX = 1000.0

_PATH_BLK = 1024
_LINK_BLK = 256


_NW = NUM_LINKS // 32


_OD_BLK = 256


def _pass1_body(x_ref, a_ref, b_ref, d_ref, sid_ref,
                t_ref, u_ref, dp_ref, m_ref):
    j = pl.program_id(0)
    nsteps = pl.num_programs(0)
    base = 1.0 + a_ref[...] * (x_ref[...] * (1.0 / X_MAX))
    t = T_MIN * base ** b_ref[...]
    t_ref[...] = t
    d = d_ref[...]
    u = jnp.dot(t, d, preferred_element_type=jnp.float32)
    u_ref[...] = u
    acc = jnp.zeros((_NW, d.shape[1]), jnp.int32)
    for k in range(32):
        bits = (d[k * _NW:(k + 1) * _NW, :] != 0.0).astype(jnp.int32)
        acc = acc | (bits << k)
    dp_ref[...] = acc
    @pl.when(j == 0)
    def _init():
        m_ref[...] = jnp.full_like(m_ref, _NEG)

    sid = sid_ref[...]
    for k in range(NUM_OD // _OD_BLK):
        od = (k * _OD_BLK
              + jax.lax.broadcasted_iota(jnp.int32, (_OD_BLK, _PATH_BLK), 0))
        vals = jnp.where(sid == od, u, _NEG)
        m = jnp.max(vals, axis=1, keepdims=True)
        m_ref[pl.ds(k * _OD_BLK, _OD_BLK), :] = jnp.maximum(
            m_ref[pl.ds(k * _OD_BLK, _OD_BLK), :], m)


def _pass1(x_hat, alpha, beta, D, segment_ids):
    grid = NUM_PATHS // _PATH_BLK
    t2, u2, dpack, segmax = pl.pallas_call(
        _pass1_body,
        grid=(grid,),
        in_specs=[
            pl.BlockSpec((1, NUM_LINKS), lambda j: (0, 0)),
            pl.BlockSpec((1, NUM_LINKS), lambda j: (0, 0)),
            pl.BlockSpec((1, NUM_LINKS), lambda j: (0, 0)),
            pl.BlockSpec((NUM_LINKS, _PATH_BLK), lambda j: (0, j)),
            pl.BlockSpec((1, _PATH_BLK), lambda j: (0, j)),
        ],
        out_specs=[
            pl.BlockSpec((1, NUM_LINKS), lambda j: (0, 0)),
            pl.BlockSpec((1, _PATH_BLK), lambda j: (0, j)),
            pl.BlockSpec((_NW, _PATH_BLK), lambda j: (0, j)),
            pl.BlockSpec((NUM_OD, 1), lambda j: (0, 0)),
        ],
        out_shape=[
            jax.ShapeDtypeStruct((1, NUM_LINKS), jnp.float32),
            jax.ShapeDtypeStruct((1, NUM_PATHS), jnp.float32),
            jax.ShapeDtypeStruct((_NW, NUM_PATHS), jnp.int32),
            jax.ShapeDtypeStruct((NUM_OD, 1), jnp.float32),
        ],
    )(x_hat.reshape(1, NUM_LINKS), alpha.reshape(1, NUM_LINKS),
      beta.reshape(1, NUM_LINKS), D, segment_ids.reshape(1, NUM_PATHS))
    return (t2.reshape(NUM_LINKS), u2.reshape(NUM_PATHS), dpack,
            segmax.reshape(NUM_OD))


def _pass2_body(dp_ref, f_ref, x_ref):
    j = pl.program_id(0)

    @pl.when(j == 0)
    def _init():
        x_ref[...] = jnp.zeros_like(x_ref)

    dp = dp_ref[...]
    fch = f_ref[...]
    for k in range(32):
        mask = ((dp >> k) & 1).astype(jnp.float32)
        x_ref[k * _NW:(k + 1) * _NW, :] += jnp.dot(
            mask, fch, preferred_element_type=jnp.float32)


def _pass2(dpack, f):
    grid = NUM_PATHS // _PATH_BLK
    x2 = pl.pallas_call(
        _pass2_body,
        grid=(grid,),
        in_specs=[
            pl.BlockSpec((_NW, _PATH_BLK), lambda j: (0, j)),
            pl.BlockSpec((_PATH_BLK, 1), lambda j: (j, 0)),
        ],
        out_specs=pl.BlockSpec((NUM_LINKS, 1), lambda j: (0, 0)),
        out_shape=jax.ShapeDtypeStruct((NUM_LINKS, 1), jnp.float32),
    )(dpack, f.reshape(NUM_PATHS, 1))
    return x2.reshape(NUM_LINKS)


_NSUB = 16
_SLICE = NUM_PATHS // _NSUB
_ROWS = _SLICE // 128
_SEG_PER_SUB = NUM_OD // _NSUB
_NEG = -3.0e38


def _seg_softmax_body(u_hbm, sid_hbm, q_hbm, m_hbm, p_hbm, f_hbm,
                      u_t, sid_t, m_t, e_t, s_t, q_t, p_t, f_t, zero_t,
                      sem, sh_sum):
    cid = lax.axis_index("c")
    sub = lax.axis_index("s")
    base = sub * _ROWS

    pltpu.sync_copy(u_hbm.at[pl.ds(base, _ROWS)], u_t)
    pltpu.sync_copy(sid_hbm.at[pl.ds(base, _ROWS)], sid_t)

    for i in range(_SEG_PER_SUB // 16):
        zero_t[pl.ds(i * 16, 16)] = jnp.zeros((16,), jnp.float32)
    pltpu.sync_copy(zero_t, sh_sum.at[pl.ds(sub * _SEG_PER_SUB,
                                            _SEG_PER_SUB)])

    for r in range(_ROWS):
        pltpu.async_copy(m_hbm.at[sid_t.at[r]], m_t.at[r], sem).wait()

    plsc.subcore_barrier()

    for r in range(_ROWS):
        for i in range(128 // 16):
            sl = pl.ds(i * 16, 16)
            e_t[r, sl] = jnp.exp(u_t[r, sl] - m_t[r, sl])
    for r in range(_ROWS):
        pltpu.sync_copy(e_t.at[r], sh_sum.at[sid_t.at[r]], add=True)

    plsc.subcore_barrier()

    off = cid * (_ROWS // 2)
    for r in range(_ROWS // 2):
        pltpu.async_copy(sh_sum.at[sid_t.at[off + r]], s_t.at[r], sem).wait()
        pltpu.async_copy(q_hbm.at[sid_t.at[off + r]], q_t.at[r], sem).wait()
    for r in range(_ROWS // 2):
        for i in range(128 // 16):
            sl = pl.ds(i * 16, 16)
            p = e_t[off + r, sl] / s_t[r, sl]
            p_t[r, sl] = p
            f_t[r, sl] = q_t[r, sl] * p
    out_row = base + off
    pltpu.sync_copy(p_t, p_hbm.at[pl.ds(out_row, _ROWS // 2)])
    pltpu.sync_copy(f_t, f_hbm.at[pl.ds(out_row, _ROWS // 2)])


def _seg_softmax(u, segment_ids, q_hat, segmax):
    fn = functools.partial(
        pl.kernel,
        out_type=[
            jax.ShapeDtypeStruct((NUM_PATHS // 128, 128), jnp.float32),
            jax.ShapeDtypeStruct((NUM_PATHS // 128, 128), jnp.float32),
        ],
        mesh=plsc.VectorSubcoreMesh(core_axis_name="c", subcore_axis_name="s"),
        scratch_types=[
            pltpu.VMEM((_ROWS, 128), jnp.float32),
            pltpu.VMEM((_ROWS, 128), jnp.int32),
            pltpu.VMEM((_ROWS, 128), jnp.float32),
            pltpu.VMEM((_ROWS, 128), jnp.float32),
            pltpu.VMEM((_ROWS // 2, 128), jnp.float32),
            pltpu.VMEM((_ROWS // 2, 128), jnp.float32),
            pltpu.VMEM((_ROWS // 2, 128), jnp.float32),
            pltpu.VMEM((_ROWS // 2, 128), jnp.float32),
            pltpu.VMEM((_SEG_PER_SUB,), jnp.float32),
            pltpu.SemaphoreType.DMA,
            pltpu.VMEM_SHARED((NUM_OD,), jnp.float32),
        ],
    )(_seg_softmax_body)
    p2, f2 = fn(u.reshape(NUM_PATHS // 128, 128),
                segment_ids.reshape(NUM_PATHS // 128, 128),
                q_hat, segmax)
    return p2.reshape(NUM_PATHS), f2.reshape(NUM_PATHS)


def kernel(x_hat, alpha, beta, q_hat, D, M, segment_ids):
    t, u, dpack, segmax = _pass1(x_hat, alpha, beta, D, segment_ids)
    p, f = _seg_softmax(u, segment_ids, q_hat, segmax)
    x = _pass2(dpack, f)
    return (x, t, f, p)


# speedup vs baseline: 3.5166x; 1.0270x over previous
import functools

import jax
import jax.numpy as jnp
from jax import lax
from jax.experimental import pallas as pl
from jax.experimental.pallas import tpu as pltpu
from jax.experimental.pallas import tpu_sc as plsc

NUM_LINKS = 4096
NUM_PATHS = 8192
NUM_OD = 2048
T_MIN = 1.0
X_MAX = 1000.0

_PATH_BLK = 1024
_LINK_BLK = 256


_NW = NUM_LINKS // 32


_OD_BLK = 256


def _pass1_body(x_ref, a_ref, b_ref, d_ref, sid_ref,
                t_ref, u_ref, dp_ref, m_ref):
    j = pl.program_id(0)
    nsteps = pl.num_programs(0)
    base = 1.0 + a_ref[...] * (x_ref[...] * (1.0 / X_MAX))
    t = T_MIN * base ** b_ref[...]
    t_ref[...] = t
    d = d_ref[...]
    u = jnp.dot(t, d, preferred_element_type=jnp.float32)
    u_ref[...] = u
    acc = jnp.zeros((_NW, d.shape[1]), jnp.int32)
    for k in range(32):
        bits = (d[k * _NW:(k + 1) * _NW, :] != 0.0).astype(jnp.int32)
        acc = acc | (bits << k)
    dp_ref[...] = acc
    @pl.when(j == 0)
    def _init():
        m_ref[...] = jnp.full_like(m_ref, _NEG)

    sid = sid_ref[...]
    for k in range(NUM_OD // _OD_BLK):
        od = (k * _OD_BLK
              + jax.lax.broadcasted_iota(jnp.int32, (_OD_BLK, _PATH_BLK), 0))
        vals = jnp.where(sid == od, u, _NEG)
        m = jnp.max(vals, axis=1, keepdims=True)
        m_ref[pl.ds(k * _OD_BLK, _OD_BLK), :] = jnp.maximum(
            m_ref[pl.ds(k * _OD_BLK, _OD_BLK), :], m)


def _pass1(x_hat, alpha, beta, D, segment_ids):
    grid = NUM_PATHS // _PATH_BLK
    t2, u2, dpack, segmax = pl.pallas_call(
        _pass1_body,
        grid=(grid,),
        in_specs=[
            pl.BlockSpec((1, NUM_LINKS), lambda j: (0, 0)),
            pl.BlockSpec((1, NUM_LINKS), lambda j: (0, 0)),
            pl.BlockSpec((1, NUM_LINKS), lambda j: (0, 0)),
            pl.BlockSpec((NUM_LINKS, _PATH_BLK), lambda j: (0, j)),
            pl.BlockSpec((1, _PATH_BLK), lambda j: (0, j)),
        ],
        out_specs=[
            pl.BlockSpec((1, NUM_LINKS), lambda j: (0, 0)),
            pl.BlockSpec((1, _PATH_BLK), lambda j: (0, j)),
            pl.BlockSpec((_NW, _PATH_BLK), lambda j: (0, j)),
            pl.BlockSpec((NUM_OD, 1), lambda j: (0, 0)),
        ],
        out_shape=[
            jax.ShapeDtypeStruct((1, NUM_LINKS), jnp.float32),
            jax.ShapeDtypeStruct((1, NUM_PATHS), jnp.float32),
            jax.ShapeDtypeStruct((_NW, NUM_PATHS), jnp.int32),
            jax.ShapeDtypeStruct((NUM_OD, 1), jnp.float32),
        ],
    )(x_hat.reshape(1, NUM_LINKS), alpha.reshape(1, NUM_LINKS),
      beta.reshape(1, NUM_LINKS), D, segment_ids.reshape(1, NUM_PATHS))
    return (t2.reshape(NUM_LINKS), u2.reshape(NUM_PATHS), dpack,
            segmax.reshape(NUM_OD))


_P2_BLK = 2048


def _pass2_body(dp_ref, f_ref, x_ref):
    j = pl.program_id(0)

    @pl.when(j == 0)
    def _init():
        x_ref[...] = jnp.zeros_like(x_ref)

    dp = dp_ref[...]
    fch = f_ref[...]
    for k in range(32):
        mask = ((dp >> k) & 1).astype(jnp.float32)
        x_ref[k * _NW:(k + 1) * _NW, :] += jnp.dot(
            mask, fch, preferred_element_type=jnp.float32)


def _pass2(dpack, f):
    grid = NUM_PATHS // _P2_BLK
    x2 = pl.pallas_call(
        _pass2_body,
        grid=(grid,),
        in_specs=[
            pl.BlockSpec((_NW, _P2_BLK), lambda j: (0, j)),
            pl.BlockSpec((_P2_BLK, 1), lambda j: (j, 0)),
        ],
        out_specs=pl.BlockSpec((NUM_LINKS, 1), lambda j: (0, 0)),
        out_shape=jax.ShapeDtypeStruct((NUM_LINKS, 1), jnp.float32),
    )(dpack, f.reshape(NUM_PATHS, 1))
    return x2.reshape(NUM_LINKS)


_NSUB = 16
_SLICE = NUM_PATHS // _NSUB
_ROWS = _SLICE // 128
_SEG_PER_SUB = NUM_OD // _NSUB
_NEG = -3.0e38


def _seg_softmax_body(u_hbm, sid_hbm, q_hbm, m_hbm, p_hbm, f_hbm,
                      u_t, sid_t, m_t, e_t, s_t, q_t, p_t, f_t, zero_t,
                      sem, sh_sum):
    cid = lax.axis_index("c")
    sub = lax.axis_index("s")
    base = sub * _ROWS

    pltpu.sync_copy(u_hbm.at[pl.ds(base, _ROWS)], u_t)
    pltpu.sync_copy(sid_hbm.at[pl.ds(base, _ROWS)], sid_t)

    for i in range(_SEG_PER_SUB // 16):
        zero_t[pl.ds(i * 16, 16)] = jnp.zeros((16,), jnp.float32)
    pltpu.sync_copy(zero_t, sh_sum.at[pl.ds(sub * _SEG_PER_SUB,
                                            _SEG_PER_SUB)])

    for r in range(_ROWS):
        pltpu.async_copy(m_hbm.at[sid_t.at[r]], m_t.at[r], sem).wait()

    plsc.subcore_barrier()

    for r in range(_ROWS):
        for i in range(128 // 16):
            sl = pl.ds(i * 16, 16)
            e_t[r, sl] = jnp.exp(u_t[r, sl] - m_t[r, sl])
    for r in range(_ROWS):
        pltpu.sync_copy(e_t.at[r], sh_sum.at[sid_t.at[r]], add=True)

    plsc.subcore_barrier()

    off = cid * (_ROWS // 2)
    for r in range(_ROWS // 2):
        pltpu.async_copy(sh_sum.at[sid_t.at[off + r]], s_t.at[r], sem).wait()
        pltpu.async_copy(q_hbm.at[sid_t.at[off + r]], q_t.at[r], sem).wait()
    for r in range(_ROWS // 2):
        for i in range(128 // 16):
            sl = pl.ds(i * 16, 16)
            p = e_t[off + r, sl] / s_t[r, sl]
            p_t[r, sl] = p
            f_t[r, sl] = q_t[r, sl] * p
    out_row = base + off
    pltpu.sync_copy(p_t, p_hbm.at[pl.ds(out_row, _ROWS // 2)])
    pltpu.sync_copy(f_t, f_hbm.at[pl.ds(out_row, _ROWS // 2)])


def _seg_softmax(u, segment_ids, q_hat, segmax):
    fn = functools.partial(
        pl.kernel,
        out_type=[
            jax.ShapeDtypeStruct((NUM_PATHS // 128, 128), jnp.float32),
            jax.ShapeDtypeStruct((NUM_PATHS // 128, 128), jnp.float32),
        ],
        mesh=plsc.VectorSubcoreMesh(core_axis_name="c", subcore_axis_name="s"),
        scratch_types=[
            pltpu.VMEM((_ROWS, 128), jnp.float32),
            pltpu.VMEM((_ROWS, 128), jnp.int32),
            pltpu.VMEM((_ROWS, 128), jnp.float32),
            pltpu.VMEM((_ROWS, 128), jnp.float32),
            pltpu.VMEM((_ROWS // 2, 128), jnp.float32),
            pltpu.VMEM((_ROWS // 2, 128), jnp.float32),
            pltpu.VMEM((_ROWS // 2, 128), jnp.float32),
            pltpu.VMEM((_ROWS // 2, 128), jnp.float32),
            pltpu.VMEM((_SEG_PER_SUB,), jnp.float32),
            pltpu.SemaphoreType.DMA,
            pltpu.VMEM_SHARED((NUM_OD,), jnp.float32),
        ],
    )(_seg_softmax_body)
    p2, f2 = fn(u.reshape(NUM_PATHS // 128, 128),
                segment_ids.reshape(NUM_PATHS // 128, 128),
                q_hat, segmax)
    return p2.reshape(NUM_PATHS), f2.reshape(NUM_PATHS)


def kernel(x_hat, alpha, beta, q_hat, D, M, segment_ids):
    t, u, dpack, segmax = _pass1(x_hat, alpha, beta, D, segment_ids)
    p, f = _seg_softmax(u, segment_ids, q_hat, segmax)
    x = _pass2(dpack, f)
    return (x, t, f, p)


# speedup vs baseline: 3.9225x; 1.1154x over previous
import functools

import jax
import jax.numpy as jnp
from jax import lax
from jax.experimental import pallas as pl
from jax.experimental.pallas import tpu as pltpu
from jax.experimental.pallas import tpu_sc as plsc

NUM_LINKS = 4096
NUM_PATHS = 8192
NUM_OD = 2048
T_MIN = 1.0
X_MAX = 1000.0

_PATH_BLK = 1024
_LINK_BLK = 256


_NW = NUM_LINKS // 32


_OD_BLK = 256


def _pass1_body(x_ref, a_ref, b_ref, d_ref, sid_ref,
                t_ref, u_ref, dp_ref, m_ref):
    j = pl.program_id(0)
    xh = jnp.reshape(x_ref[...], (1, NUM_LINKS))
    al = jnp.reshape(a_ref[...], (1, NUM_LINKS))
    be = jnp.reshape(b_ref[...], (1, NUM_LINKS))
    base = 1.0 + al * (xh * (1.0 / X_MAX))
    t = T_MIN * base ** be
    t_ref[...] = jnp.reshape(t, (NUM_LINKS // 128, 128))
    d = d_ref[...]
    u = jnp.dot(t, d, preferred_element_type=jnp.float32)
    u_ref[...] = jnp.reshape(u, (_PATH_BLK // 128, 128))
    acc = jnp.zeros((_NW, d.shape[1]), jnp.int32)
    for k in range(32):
        bits = (d[k * _NW:(k + 1) * _NW, :] != 0.0).astype(jnp.int32)
        acc = acc | (bits << k)
    dp_ref[...] = acc
    @pl.when(j == 0)
    def _init():
        m_ref[...] = jnp.full_like(m_ref, _NEG)

    sid = jnp.reshape(sid_ref[...], (1, _PATH_BLK))
    for k in range(NUM_OD // _OD_BLK):
        od = (k * _OD_BLK
              + jax.lax.broadcasted_iota(jnp.int32, (_OD_BLK, _PATH_BLK), 0))
        vals = jnp.where(sid == od, u, _NEG)
        m = jnp.max(vals, axis=1, keepdims=True)
        m2 = jnp.reshape(m, (_OD_BLK // 128, 128))
        sl = pl.ds(k * (_OD_BLK // 128), _OD_BLK // 128)
        m_ref[sl, :] = jnp.maximum(m_ref[sl, :], m2)


def _pass1(x_hat, alpha, beta, D, segment_ids):
    grid = NUM_PATHS // _PATH_BLK
    t2, u2, dpack, segmax = pl.pallas_call(
        _pass1_body,
        grid=(grid,),
        in_specs=[
            pl.BlockSpec((NUM_LINKS // 128, 128), lambda j: (0, 0)),
            pl.BlockSpec((NUM_LINKS // 128, 128), lambda j: (0, 0)),
            pl.BlockSpec((NUM_LINKS // 128, 128), lambda j: (0, 0)),
            pl.BlockSpec((NUM_LINKS, _PATH_BLK), lambda j: (0, j)),
            pl.BlockSpec((_PATH_BLK // 128, 128), lambda j: (j, 0)),
        ],
        out_specs=[
            pl.BlockSpec((NUM_LINKS // 128, 128), lambda j: (0, 0)),
            pl.BlockSpec((_PATH_BLK // 128, 128), lambda j: (j, 0)),
            pl.BlockSpec((_NW, _PATH_BLK), lambda j: (0, j)),
            pl.BlockSpec((NUM_OD // 128, 128), lambda j: (0, 0)),
        ],
        out_shape=[
            jax.ShapeDtypeStruct((NUM_LINKS // 128, 128), jnp.float32),
            jax.ShapeDtypeStruct((NUM_PATHS // 128, 128), jnp.float32),
            jax.ShapeDtypeStruct((_NW, NUM_PATHS), jnp.int32),
            jax.ShapeDtypeStruct((NUM_OD // 128, 128), jnp.float32),
        ],
    )(x_hat.reshape(NUM_LINKS // 128, 128),
      alpha.reshape(NUM_LINKS // 128, 128),
      beta.reshape(NUM_LINKS // 128, 128), D,
      segment_ids.reshape(NUM_PATHS // 128, 128))
    return (t2.reshape(NUM_LINKS), u2, dpack, segmax.reshape(NUM_OD))


_P2_BLK = 2048


def _pass2_body(dp_ref, f_ref, x_ref):
    j = pl.program_id(0)

    @pl.when(j == 0)
    def _init():
        x_ref[...] = jnp.zeros_like(x_ref)

    dp = dp_ref[...]
    fch = jnp.reshape(f_ref[...], (1, _P2_BLK))
    for k in range(32):
        mask = ((dp >> k) & 1).astype(jnp.float32)
        xk = lax.dot_general(fch, mask, (((1,), (1,)), ((), ())),
                             preferred_element_type=jnp.float32)
        x_ref[pl.ds(k, 1), :] += xk


def _pass2(dpack, f):
    grid = NUM_PATHS // _P2_BLK
    x2 = pl.pallas_call(
        _pass2_body,
        grid=(grid,),
        in_specs=[
            pl.BlockSpec((_NW, _P2_BLK), lambda j: (0, j)),
            pl.BlockSpec((_P2_BLK // 128, 128), lambda j: (j, 0)),
        ],
        out_specs=pl.BlockSpec((32, _NW), lambda j: (0, 0)),
        out_shape=jax.ShapeDtypeStruct((32, _NW), jnp.float32),
    )(dpack, f)
    return x2.reshape(NUM_LINKS)


_NSUB = 16
_SLICE = NUM_PATHS // _NSUB
_ROWS = _SLICE // 128
_SEG_PER_SUB = NUM_OD // _NSUB
_NEG = -3.0e38


def _seg_softmax_body(u_hbm, sid_hbm, q_hbm, m_hbm, p_hbm, f_hbm,
                      u_t, sid_t, m_t, e_t, s_t, q_t, p_t, f_t, zero_t,
                      sem, sh_sum):
    cid = lax.axis_index("c")
    sub = lax.axis_index("s")
    base = sub * _ROWS

    pltpu.sync_copy(u_hbm.at[pl.ds(base, _ROWS)], u_t)
    pltpu.sync_copy(sid_hbm.at[pl.ds(base, _ROWS)], sid_t)

    for i in range(_SEG_PER_SUB // 16):
        zero_t[pl.ds(i * 16, 16)] = jnp.zeros((16,), jnp.float32)
    pltpu.sync_copy(zero_t, sh_sum.at[pl.ds(sub * _SEG_PER_SUB,
                                            _SEG_PER_SUB)])

    for r in range(_ROWS):
        pltpu.async_copy(m_hbm.at[sid_t.at[r]], m_t.at[r], sem).wait()

    plsc.subcore_barrier()

    for r in range(_ROWS):
        for i in range(128 // 16):
            sl = pl.ds(i * 16, 16)
            e_t[r, sl] = jnp.exp(u_t[r, sl] - m_t[r, sl])
    for r in range(_ROWS):
        pltpu.sync_copy(e_t.at[r], sh_sum.at[sid_t.at[r]], add=True)

    plsc.subcore_barrier()

    off = cid * (_ROWS // 2)
    for r in range(_ROWS // 2):
        pltpu.async_copy(sh_sum.at[sid_t.at[off + r]], s_t.at[r], sem).wait()
        pltpu.async_copy(q_hbm.at[sid_t.at[off + r]], q_t.at[r], sem).wait()
    for r in range(_ROWS // 2):
        for i in range(128 // 16):
            sl = pl.ds(i * 16, 16)
            p = e_t[off + r, sl] / s_t[r, sl]
            p_t[r, sl] = p
            f_t[r, sl] = q_t[r, sl] * p
    out_row = base + off
    pltpu.sync_copy(p_t, p_hbm.at[pl.ds(out_row, _ROWS // 2)])
    pltpu.sync_copy(f_t, f_hbm.at[pl.ds(out_row, _ROWS // 2)])


def _seg_softmax(u, segment_ids, q_hat, segmax):
    fn = functools.partial(
        pl.kernel,
        out_type=[
            jax.ShapeDtypeStruct((NUM_PATHS // 128, 128), jnp.float32),
            jax.ShapeDtypeStruct((NUM_PATHS // 128, 128), jnp.float32),
        ],
        mesh=plsc.VectorSubcoreMesh(core_axis_name="c", subcore_axis_name="s"),
        scratch_types=[
            pltpu.VMEM((_ROWS, 128), jnp.float32),
            pltpu.VMEM((_ROWS, 128), jnp.int32),
            pltpu.VMEM((_ROWS, 128), jnp.float32),
            pltpu.VMEM((_ROWS, 128), jnp.float32),
            pltpu.VMEM((_ROWS // 2, 128), jnp.float32),
            pltpu.VMEM((_ROWS // 2, 128), jnp.float32),
            pltpu.VMEM((_ROWS // 2, 128), jnp.float32),
            pltpu.VMEM((_ROWS // 2, 128), jnp.float32),
            pltpu.VMEM((_SEG_PER_SUB,), jnp.float32),
            pltpu.SemaphoreType.DMA,
            pltpu.VMEM_SHARED((NUM_OD,), jnp.float32),
        ],
    )(_seg_softmax_body)
    p2, f2 = fn(u, segment_ids.reshape(NUM_PATHS // 128, 128), q_hat, segmax)
    return p2, f2


def kernel(x_hat, alpha, beta, q_hat, D, M, segment_ids):
    t, u, dpack, segmax = _pass1(x_hat, alpha, beta, D, segment_ids)
    p, f = _seg_softmax(u, segment_ids, q_hat, segmax)
    x = _pass2(dpack, f)
    return (x, t, f.reshape(NUM_PATHS), p.reshape(NUM_PATHS))


# speedup vs baseline: 3.9312x; 1.0022x over previous
import functools

import jax
import jax.numpy as jnp
from jax import lax
from jax.experimental import pallas as pl
from jax.experimental.pallas import tpu as pltpu
from jax.experimental.pallas import tpu_sc as plsc

NUM_LINKS = 4096
NUM_PATHS = 8192
NUM_OD = 2048
T_MIN = 1.0
X_MAX = 1000.0

_PATH_BLK = 1024
_LINK_BLK = 256


_NW = NUM_LINKS // 32


_OD_BLK = 256


def _pass1_body(x_ref, a_ref, b_ref, d_ref, sid_ref,
                t_ref, u_ref, dp_ref, m_ref):
    j = pl.program_id(0)
    xh = jnp.reshape(x_ref[...], (1, NUM_LINKS))
    al = jnp.reshape(a_ref[...], (1, NUM_LINKS))
    be = jnp.reshape(b_ref[...], (1, NUM_LINKS))
    base = 1.0 + al * (xh * (1.0 / X_MAX))
    t = T_MIN * base ** be
    t_ref[...] = jnp.reshape(t, (NUM_LINKS // 128, 128))
    d = d_ref[...]
    u = jnp.dot(t, d, preferred_element_type=jnp.float32)
    u_ref[...] = jnp.reshape(u, (_PATH_BLK // 128, 128))
    acc = jnp.zeros((_NW, d.shape[1]), jnp.int32)
    for k in range(32):
        bits = (d[k * _NW:(k + 1) * _NW, :] != 0.0).astype(jnp.int32)
        acc = acc | (bits << k)
    dp_ref[...] = acc
    @pl.when(j == 0)
    def _init():
        m_ref[...] = jnp.full_like(m_ref, _NEG)

    sid = jnp.reshape(sid_ref[...], (1, _PATH_BLK))
    for k in range(NUM_OD // _OD_BLK):
        od = (k * _OD_BLK
              + jax.lax.broadcasted_iota(jnp.int32, (_OD_BLK, _PATH_BLK), 0))
        vals = jnp.where(sid == od, u, _NEG)
        m = jnp.max(vals, axis=1, keepdims=True)
        m2 = jnp.reshape(m, (_OD_BLK // 128, 128))
        sl = pl.ds(k * (_OD_BLK // 128), _OD_BLK // 128)
        m_ref[sl, :] = jnp.maximum(m_ref[sl, :], m2)


def _pass1(x_hat, alpha, beta, D, segment_ids):
    grid = NUM_PATHS // _PATH_BLK
    t2, u2, dpack, segmax = pl.pallas_call(
        _pass1_body,
        grid=(grid,),
        in_specs=[
            pl.BlockSpec((NUM_LINKS // 128, 128), lambda j: (0, 0)),
            pl.BlockSpec((NUM_LINKS // 128, 128), lambda j: (0, 0)),
            pl.BlockSpec((NUM_LINKS // 128, 128), lambda j: (0, 0)),
            pl.BlockSpec((NUM_LINKS, _PATH_BLK), lambda j: (0, j)),
            pl.BlockSpec((_PATH_BLK // 128, 128), lambda j: (j, 0)),
        ],
        out_specs=[
            pl.BlockSpec((NUM_LINKS // 128, 128), lambda j: (0, 0)),
            pl.BlockSpec((_PATH_BLK // 128, 128), lambda j: (j, 0)),
            pl.BlockSpec((_NW, _PATH_BLK), lambda j: (0, j)),
            pl.BlockSpec((NUM_OD // 128, 128), lambda j: (0, 0)),
        ],
        out_shape=[
            jax.ShapeDtypeStruct((NUM_LINKS // 128, 128), jnp.float32),
            jax.ShapeDtypeStruct((NUM_PATHS // 128, 128), jnp.float32),
            jax.ShapeDtypeStruct((_NW, NUM_PATHS), jnp.int32),
            jax.ShapeDtypeStruct((NUM_OD // 128, 128), jnp.float32),
        ],
    )(x_hat.reshape(NUM_LINKS // 128, 128),
      alpha.reshape(NUM_LINKS // 128, 128),
      beta.reshape(NUM_LINKS // 128, 128), D,
      segment_ids.reshape(NUM_PATHS // 128, 128))
    return (t2.reshape(NUM_LINKS), u2, dpack, segmax.reshape(NUM_OD))


_P2_BLK = 2048


def _pass2_body(dp_ref, f_ref, x_ref):
    j = pl.program_id(0)

    @pl.when(j == 0)
    def _init():
        x_ref[...] = jnp.zeros_like(x_ref)

    dp = dp_ref[...]
    fch = jnp.reshape(f_ref[...], (1, _P2_BLK))
    for k in range(32):
        mask = ((dp >> k) & 1).astype(jnp.float32)
        xk = lax.dot_general(fch, mask, (((1,), (1,)), ((), ())),
                             preferred_element_type=jnp.float32)
        x_ref[pl.ds(k, 1), :] += xk


def _pass2(dpack, f):
    grid = NUM_PATHS // _P2_BLK
    x2 = pl.pallas_call(
        _pass2_body,
        grid=(grid,),
        in_specs=[
            pl.BlockSpec((_NW, _P2_BLK), lambda j: (0, j)),
            pl.BlockSpec((_P2_BLK // 128, 128), lambda j: (j, 0)),
        ],
        out_specs=pl.BlockSpec((32, _NW), lambda j: (0, 0)),
        out_shape=jax.ShapeDtypeStruct((32, _NW), jnp.float32),
    )(dpack, f)
    return x2.reshape(NUM_LINKS)


_NSUB = 16
_SLICE = NUM_PATHS // _NSUB
_ROWS = _SLICE // 128
_SEG_PER_SUB = NUM_OD // _NSUB
_NEG = -3.0e38


def _seg_softmax_body(u_hbm, sid_hbm, q_hbm, m_hbm, p_hbm, f_hbm,
                      u_t, sid_t, m_t, e_t, s_t, q_t, p_t, f_t, zero_t,
                      sem0, sem1, sem2, sem3, sh_sum):
    sems = (sem0, sem1, sem2, sem3)
    cid = lax.axis_index("c")
    sub = lax.axis_index("s")
    base = sub * _ROWS

    pltpu.sync_copy(u_hbm.at[pl.ds(base, _ROWS)], u_t)
    pltpu.sync_copy(sid_hbm.at[pl.ds(base, _ROWS)], sid_t)

    for i in range(_SEG_PER_SUB // 16):
        zero_t[pl.ds(i * 16, 16)] = jnp.zeros((16,), jnp.float32)
    pltpu.sync_copy(zero_t, sh_sum.at[pl.ds(sub * _SEG_PER_SUB,
                                            _SEG_PER_SUB)])

    copies = [pltpu.async_copy(m_hbm.at[sid_t.at[r]], m_t.at[r], sems[r])
              for r in range(_ROWS)]
    for c in copies:
        c.wait()

    plsc.subcore_barrier()

    for r in range(_ROWS):
        for i in range(128 // 16):
            sl = pl.ds(i * 16, 16)
            e_t[r, sl] = jnp.exp(u_t[r, sl] - m_t[r, sl])
    for r in range(_ROWS):
        pltpu.sync_copy(e_t.at[r], sh_sum.at[sid_t.at[r]], add=True)

    plsc.subcore_barrier()

    off = cid * (_ROWS // 2)
    copies = [pltpu.async_copy(sh_sum.at[sid_t.at[off + r]], s_t.at[r],
                               sems[r]) for r in range(_ROWS // 2)]
    copies += [pltpu.async_copy(q_hbm.at[sid_t.at[off + r]], q_t.at[r],
                                sems[2 + r]) for r in range(_ROWS // 2)]
    for c in copies:
        c.wait()
    for r in range(_ROWS // 2):
        for i in range(128 // 16):
            sl = pl.ds(i * 16, 16)
            p = e_t[off + r, sl] / s_t[r, sl]
            p_t[r, sl] = p
            f_t[r, sl] = q_t[r, sl] * p
    out_row = base + off
    pltpu.sync_copy(p_t, p_hbm.at[pl.ds(out_row, _ROWS // 2)])
    pltpu.sync_copy(f_t, f_hbm.at[pl.ds(out_row, _ROWS // 2)])


def _seg_softmax(u, segment_ids, q_hat, segmax):
    fn = functools.partial(
        pl.kernel,
        out_type=[
            jax.ShapeDtypeStruct((NUM_PATHS // 128, 128), jnp.float32),
            jax.ShapeDtypeStruct((NUM_PATHS // 128, 128), jnp.float32),
        ],
        mesh=plsc.VectorSubcoreMesh(core_axis_name="c", subcore_axis_name="s"),
        scratch_types=[
            pltpu.VMEM((_ROWS, 128), jnp.float32),
            pltpu.VMEM((_ROWS, 128), jnp.int32),
            pltpu.VMEM((_ROWS, 128), jnp.float32),
            pltpu.VMEM((_ROWS, 128), jnp.float32),
            pltpu.VMEM((_ROWS // 2, 128), jnp.float32),
            pltpu.VMEM((_ROWS // 2, 128), jnp.float32),
            pltpu.VMEM((_ROWS // 2, 128), jnp.float32),
            pltpu.VMEM((_ROWS // 2, 128), jnp.float32),
            pltpu.VMEM((_SEG_PER_SUB,), jnp.float32),
            pltpu.SemaphoreType.DMA,
            pltpu.SemaphoreType.DMA,
            pltpu.SemaphoreType.DMA,
            pltpu.SemaphoreType.DMA,
            pltpu.VMEM_SHARED((NUM_OD,), jnp.float32),
        ],
    )(_seg_softmax_body)
    p2, f2 = fn(u, segment_ids.reshape(NUM_PATHS // 128, 128), q_hat, segmax)
    return p2, f2


def kernel(x_hat, alpha, beta, q_hat, D, M, segment_ids):
    t, u, dpack, segmax = _pass1(x_hat, alpha, beta, D, segment_ids)
    p, f = _seg_softmax(u, segment_ids, q_hat, segmax)
    x = _pass2(dpack, f)
    return (x, t, f.reshape(NUM_PATHS), p.reshape(NUM_PATHS))


# speedup vs baseline: 3.9400x; 1.0022x over previous
import functools

import jax
import jax.numpy as jnp
from jax import lax
from jax.experimental import pallas as pl
from jax.experimental.pallas import tpu as pltpu
from jax.experimental.pallas import tpu_sc as plsc

NUM_LINKS = 4096
NUM_PATHS = 8192
NUM_OD = 2048
T_MIN = 1.0
X_MAX = 1000.0

_PATH_BLK = 1024
_LINK_BLK = 256


_NW = NUM_LINKS // 32


_OD_BLK = 256


def _pass1_body(x_ref, a_ref, b_ref, d_ref, sid_ref,
                t_ref, u_ref, dp_ref, m_ref):
    j = pl.program_id(0)
    xh = jnp.reshape(x_ref[...], (1, NUM_LINKS))
    al = jnp.reshape(a_ref[...], (1, NUM_LINKS))
    be = jnp.reshape(b_ref[...], (1, NUM_LINKS))
    base = 1.0 + al * (xh * (1.0 / X_MAX))
    t = T_MIN * base ** be
    t_ref[...] = jnp.reshape(t, (NUM_LINKS // 128, 128))
    d = d_ref[...]
    u = jnp.dot(t, d, preferred_element_type=jnp.float32)
    u_ref[...] = jnp.reshape(u, (_PATH_BLK // 128, 128))
    acc = jnp.zeros((_NW, d.shape[1]), jnp.int32)
    for k in range(32):
        bits = (d[k * _NW:(k + 1) * _NW, :] != 0.0).astype(jnp.int32)
        acc = acc | (bits << k)
    dp_ref[...] = acc
    @pl.when(j == 0)
    def _init():
        m_ref[...] = jnp.full_like(m_ref, _NEG)

    sid = jnp.reshape(sid_ref[...], (1, _PATH_BLK))
    for k in range(NUM_OD // _OD_BLK):
        od = (k * _OD_BLK
              + jax.lax.broadcasted_iota(jnp.int32, (_OD_BLK, _PATH_BLK), 0))
        vals = jnp.where(sid == od, u, _NEG)
        m = jnp.max(vals, axis=1, keepdims=True)
        m2 = jnp.reshape(m, (_OD_BLK // 128, 128))
        sl = pl.ds(k * (_OD_BLK // 128), _OD_BLK // 128)
        m_ref[sl, :] = jnp.maximum(m_ref[sl, :], m2)


def _pass1(x_hat, alpha, beta, D, segment_ids):
    grid = NUM_PATHS // _PATH_BLK
    t2, u2, dpack, segmax = pl.pallas_call(
        _pass1_body,
        grid=(grid,),
        in_specs=[
            pl.BlockSpec((NUM_LINKS // 128, 128), lambda j: (0, 0)),
            pl.BlockSpec((NUM_LINKS // 128, 128), lambda j: (0, 0)),
            pl.BlockSpec((NUM_LINKS // 128, 128), lambda j: (0, 0)),
            pl.BlockSpec((NUM_LINKS, _PATH_BLK), lambda j: (0, j)),
            pl.BlockSpec((_PATH_BLK // 128, 128), lambda j: (j, 0)),
        ],
        out_specs=[
            pl.BlockSpec((NUM_LINKS // 128, 128), lambda j: (0, 0)),
            pl.BlockSpec((_PATH_BLK // 128, 128), lambda j: (j, 0)),
            pl.BlockSpec((_NW, _PATH_BLK), lambda j: (0, j)),
            pl.BlockSpec((NUM_OD // 128, 128), lambda j: (0, 0)),
        ],
        out_shape=[
            jax.ShapeDtypeStruct((NUM_LINKS // 128, 128), jnp.float32),
            jax.ShapeDtypeStruct((NUM_PATHS // 128, 128), jnp.float32),
            jax.ShapeDtypeStruct((_NW, NUM_PATHS), jnp.int32),
            jax.ShapeDtypeStruct((NUM_OD // 128, 128), jnp.float32),
        ],
    )(x_hat.reshape(NUM_LINKS // 128, 128),
      alpha.reshape(NUM_LINKS // 128, 128),
      beta.reshape(NUM_LINKS // 128, 128), D,
      segment_ids.reshape(NUM_PATHS // 128, 128))
    return (t2.reshape(NUM_LINKS), u2, dpack, segmax.reshape(NUM_OD))


_P2_BLK = 2048


def _pass2_body(dp_ref, f_ref, x_ref):
    j = pl.program_id(0)

    @pl.when(j == 0)
    def _init():
        x_ref[...] = jnp.zeros_like(x_ref)

    dp = dp_ref[...]
    fch = jnp.reshape(f_ref[...], (1, _P2_BLK))
    for k in range(32):
        mask = ((dp >> k) & 1).astype(jnp.float32)
        xk = lax.dot_general(fch, mask, (((1,), (1,)), ((), ())),
                             preferred_element_type=jnp.float32)
        x_ref[pl.ds(k, 1), :] += xk


def _pass2(dpack, f):
    grid = NUM_PATHS // _P2_BLK
    x2 = pl.pallas_call(
        _pass2_body,
        grid=(grid,),
        in_specs=[
            pl.BlockSpec((_NW, _P2_BLK), lambda j: (0, j)),
            pl.BlockSpec((_P2_BLK // 128, 128), lambda j: (j, 0)),
        ],
        out_specs=pl.BlockSpec((32, _NW), lambda j: (0, 0)),
        out_shape=jax.ShapeDtypeStruct((32, _NW), jnp.float32),
    )(dpack, f)
    return x2.reshape(NUM_LINKS)


_NSUB = 16
_SLICE = NUM_PATHS // _NSUB
_ROWS = _SLICE // 128
_SEG_PER_SUB = NUM_OD // _NSUB
_NEG = -3.0e38


def _seg_softmax_body(u_hbm, sid_hbm, q_hbm, m_hbm, p_hbm, f_hbm,
                      u_t, sid_t, m_t, e_t, s_t, q_t, p_t, f_t, zero_t,
                      sem0, sem1, sem2, sem3, sh_sum):
    sems = (sem0, sem1, sem2, sem3)
    cid = lax.axis_index("c")
    sub = lax.axis_index("s")
    base = sub * _ROWS

    pltpu.sync_copy(u_hbm.at[pl.ds(base, _ROWS)], u_t)
    pltpu.sync_copy(sid_hbm.at[pl.ds(base, _ROWS)], sid_t)

    for i in range(_SEG_PER_SUB // 16):
        zero_t[pl.ds(i * 16, 16)] = jnp.zeros((16,), jnp.float32)
    pltpu.sync_copy(zero_t, sh_sum.at[pl.ds(sub * _SEG_PER_SUB,
                                            _SEG_PER_SUB)])

    copies = [pltpu.async_copy(m_hbm.at[sid_t.at[r]], m_t.at[r], sems[r])
              for r in range(_ROWS)]
    for c in copies:
        c.wait()

    plsc.subcore_barrier()

    for r in range(_ROWS):
        for i in range(128 // 16):
            sl = pl.ds(i * 16, 16)
            e_t[r, sl] = jnp.exp(u_t[r, sl] - m_t[r, sl])
    copies = [pltpu.async_copy(e_t.at[r], sh_sum.at[sid_t.at[r]], sems[r],
                               add=True) for r in range(_ROWS)]
    for c in copies:
        c.wait()

    plsc.subcore_barrier()

    off = cid * (_ROWS // 2)
    copies = [pltpu.async_copy(sh_sum.at[sid_t.at[off + r]], s_t.at[r],
                               sems[r]) for r in range(_ROWS // 2)]
    copies += [pltpu.async_copy(q_hbm.at[sid_t.at[off + r]], q_t.at[r],
                                sems[2 + r]) for r in range(_ROWS // 2)]
    for c in copies:
        c.wait()
    for r in range(_ROWS // 2):
        for i in range(128 // 16):
            sl = pl.ds(i * 16, 16)
            p = e_t[off + r, sl] / s_t[r, sl]
            p_t[r, sl] = p
            f_t[r, sl] = q_t[r, sl] * p
    out_row = base + off
    pltpu.sync_copy(p_t, p_hbm.at[pl.ds(out_row, _ROWS // 2)])
    pltpu.sync_copy(f_t, f_hbm.at[pl.ds(out_row, _ROWS // 2)])


def _seg_softmax(u, segment_ids, q_hat, segmax):
    fn = functools.partial(
        pl.kernel,
        out_type=[
            jax.ShapeDtypeStruct((NUM_PATHS // 128, 128), jnp.float32),
            jax.ShapeDtypeStruct((NUM_PATHS // 128, 128), jnp.float32),
        ],
        mesh=plsc.VectorSubcoreMesh(core_axis_name="c", subcore_axis_name="s"),
        scratch_types=[
            pltpu.VMEM((_ROWS, 128), jnp.float32),
            pltpu.VMEM((_ROWS, 128), jnp.int32),
            pltpu.VMEM((_ROWS, 128), jnp.float32),
            pltpu.VMEM((_ROWS, 128), jnp.float32),
            pltpu.VMEM((_ROWS // 2, 128), jnp.float32),
            pltpu.VMEM((_ROWS // 2, 128), jnp.float32),
            pltpu.VMEM((_ROWS // 2, 128), jnp.float32),
            pltpu.VMEM((_ROWS // 2, 128), jnp.float32),
            pltpu.VMEM((_SEG_PER_SUB,), jnp.float32),
            pltpu.SemaphoreType.DMA,
            pltpu.SemaphoreType.DMA,
            pltpu.SemaphoreType.DMA,
            pltpu.SemaphoreType.DMA,
            pltpu.VMEM_SHARED((NUM_OD,), jnp.float32),
        ],
    )(_seg_softmax_body)
    p2, f2 = fn(u, segment_ids.reshape(NUM_PATHS // 128, 128), q_hat, segmax)
    return p2, f2


def kernel(x_hat, alpha, beta, q_hat, D, M, segment_ids):
    t, u, dpack, segmax = _pass1(x_hat, alpha, beta, D, segment_ids)
    p, f = _seg_softmax(u, segment_ids, q_hat, segmax)
    x = _pass2(dpack, f)
    return (x, t, f.reshape(NUM_PATHS), p.reshape(NUM_PATHS))
